# Initial kernel scaffold; baseline (speedup 1.0000x reference)
import jax
import jax.numpy as jnp
from jax.experimental import pallas as pl


def kernel(A, b, c, u):
    raise NotImplementedError("write your pallas kernel here")


# speedup vs baseline: 4.7447x; 4.7447x over previous
import functools

import jax
import jax.numpy as jnp
from jax.experimental import pallas as pl
from jax.experimental.pallas import tpu as pltpu

_THETA = 10.0
_MAX_ITER = 30


def _sigmoid(x):
    return 1.0 / (1.0 + jnp.exp(-x))


def _logaddexp0(x):
    return jnp.maximum(x, 0.0) + jnp.log1p(jnp.exp(-jnp.abs(x)))


def _solver_kernel(a_ref, b_ref, c_ref, u_ref, x_ref, eta_ref):
    b = b_ref[...]
    c = c_ref[...]
    u = u_ref[...]
    theta_u = _THETA * u
    dtype_eps = float(jnp.finfo(jnp.float32).eps)
    btb = jnp.sum(b * b, axis=-1, keepdims=True)

    m_sz, n_sz = a_ref.shape
    MC = 256
    NC = 512

    def At_mul(w):
        w16 = w.astype(jnp.bfloat16)
        acc = None
        for i in range(m_sz // MC):
            part = jax.lax.dot_general(
                w16[:, i * MC:(i + 1) * MC], a_ref[i * MC:(i + 1) * MC, :],
                (((1,), (0,)), ((), ())), preferred_element_type=jnp.float32)
            acc = part if acc is None else acc + part
        return acc

    def A_mul(v):
        v16 = v.astype(jnp.bfloat16)
        acc = None
        for j in range(n_sz // NC):
            part = jax.lax.dot_general(
                v16[:, j * NC:(j + 1) * NC], a_ref[:, j * NC:(j + 1) * NC],
                (((1,), (1,)), ((), ())), preferred_element_type=jnp.float32)
            acc = part if acc is None else acc + part
        return acc

    m = b.shape[-1]
    n = c.shape[-1]
    M = jnp.full((1, 1), _THETA, dtype=jnp.float32)
    beta_old = jnp.zeros((1, 1), dtype=jnp.float32)
    last_cond = jnp.zeros((1, 1), dtype=jnp.float32)
    eta = jnp.zeros((1, m), dtype=jnp.float32)
    zeta = jnp.zeros((1, m), dtype=jnp.float32)
    p_eta = jnp.zeros((1, n), dtype=jnp.float32)
    p_zeta = jnp.zeros((1, n), dtype=jnp.float32)
    x_final_pu = _sigmoid(-c * theta_u)

    def body(_, carry):
        (M, beta_old, last_cond, eta, zeta, p_eta, p_zeta, x_final_pu) = carry
        alpha = 0.5 / M + jnp.sqrt((0.25 / M + beta_old) / M)
        beta_new = beta_old + alpha
        tau = alpha / beta_new
        p_lam = p_eta + tau * (p_zeta - p_eta)
        neg_l = -(c - p_lam) * theta_u
        x_lam = _sigmoid(neg_l)
        q = A_mul(u * x_lam)
        grad = q - b
        zeta_new = zeta - alpha * grad
        eta_new = eta + tau * (zeta_new - eta)
        t = At_mul(grad)
        p_zeta_new = p_zeta - alpha * t
        p_eta_new = p_eta + tau * (p_zeta_new - p_eta)
        neg_e = -(c - p_eta_new) * theta_u
        gap = (jnp.sum(q * q, axis=-1, keepdims=True) - btb) * (0.5 / M) + (
            jnp.sum(_logaddexp0(neg_e) - _logaddexp0(neg_l),
                    axis=-1, keepdims=True) / _THETA)
        cond = (gap <= dtype_eps).astype(jnp.float32)
        cond_b = cond > 0.5
        M = jnp.maximum(
            jnp.where(cond_b, jnp.where(last_cond > 0.5, M * 0.5, M), M * 2.0),
            dtype_eps)
        beta_old = jnp.where(cond_b, beta_new, beta_old)
        eta = jnp.where(cond_b, eta_new, eta)
        zeta = jnp.where(cond_b, zeta_new, zeta)
        p_eta = jnp.where(cond_b, p_eta_new, p_eta)
        p_zeta = jnp.where(cond_b, p_zeta_new, p_zeta)
        x_final_pu = jnp.where(cond_b, x_final_pu + tau * (x_lam - x_final_pu),
                               x_final_pu)
        return (M, beta_old, cond, eta, zeta, p_eta, p_zeta, x_final_pu)

    carry = (M, beta_old, last_cond, eta, zeta, p_eta, p_zeta, x_final_pu)
    carry = jax.lax.fori_loop(0, _MAX_ITER, body, carry)
    (_, _, _, eta, _, _, _, x_final_pu) = carry
    x_ref[...] = u * x_final_pu
    eta_ref[...] = eta


@jax.jit
def kernel(A, b, c, u):
    m, n = A.shape
    a_bf = A.astype(jnp.bfloat16)
    x_final, eta = pl.pallas_call(
        _solver_kernel,
        out_shape=(jax.ShapeDtypeStruct((1, n), jnp.float32),
                   jax.ShapeDtypeStruct((1, m), jnp.float32)),
        compiler_params=pltpu.CompilerParams(
            vmem_limit_bytes=100 * 1024 * 1024),
    )(a_bf, b, c, u)
    return (x_final, eta)
